# baseline (speedup 1.0000x reference)
import jax
import jax.numpy as jnp
from jax import lax
from jax.experimental import pallas as pl

B, N, D, CC, K, M, KNN = 16, 625, 128, 2, 8, 64, 32
NP = 640
INF = 3e38
PADIDX = 1000
NB = 2
BF = jnp.bfloat16


def _split(a):
    hi = a.astype(BF)
    lo = (a - hi.astype(jnp.float32)).astype(BF)
    return hi, lo


def _dot3(ah, al, bh, bl):
    f32 = jnp.float32
    return (jnp.dot(ah, bh, preferred_element_type=f32)
            + jnp.dot(ah, bl, preferred_element_type=f32)
            + jnp.dot(al, bh, preferred_element_type=f32))


def _bag_kernel(x_ref, xt_ref, xh_ref, xl_ref, xth_ref, xtl_ref, idct_ref,
                w1th_ref, w1tl_ref, w2tb_ref, with_ref, witl_ref,
                bit_ref, wih_ref, wil_ref, bi_ref, whh_ref, whl_ref, bh_ref,
                li_ref, lb_ref):
    f32 = jnp.float32
    rowi = lax.broadcasted_iota(jnp.int32, (NP, 1), 0)
    nlane128 = lax.broadcasted_iota(jnp.int32, (128, NP), 1)
    kio = lax.broadcasted_iota(jnp.int32, (1, K), 1)

    dists, rms, inits = [], [], []
    for b in range(NB):
        xb = x_ref[b]
        xt = xt_ref[b]
        sq = jnp.sum(xb * xb, axis=1, keepdims=True)
        sqm = jnp.where(rowi < N, sq, INF)
        sqT = jnp.sum(xt * xt, axis=0, keepdims=True)
        G = _dot3(xh_ref[b], xl_ref[b], xth_ref[b], xtl_ref[b])
        dist = sqm + sqT - 2.0 * G

        rm = jnp.min(dist, axis=0, keepdims=True)
        maxsq = jnp.max(jnp.where(rowi < N, sq, -INF), axis=0, keepdims=True)
        hi0 = (jnp.sqrt(sqT) + jnp.sqrt(maxsq)) ** 2 + 1.0
        dists.append(dist)
        rms.append(rm)
        inits.append((rm, hi0))

    def body(_, carry):
        out = []
        for b in range(NB):
            lo, hi = carry[b]
            mid = 0.5 * (lo + hi)
            cnt = jnp.sum((dists[b] <= mid).astype(f32), axis=0, keepdims=True)
            pred = cnt < (KNN + 1.0)
            out.append((jnp.where(pred, mid, lo), jnp.where(pred, hi, mid)))
        return tuple(out)

    finals = lax.fori_loop(0, 13, body, tuple(inits))

    for b in range(NB):
        xb, xt, dist, rm = x_ref[b], xt_ref[b], dists[b], rms[b]
        hi = finals[b][1]
        wm = ((dist <= hi) & (dist != rm)).astype(f32)
        csel = jnp.sum(wm, axis=0, keepdims=True)

        aggT = jnp.dot(xt, wm, preferred_element_type=f32) * (1.0 / csel)
        uh, ul = _split(xt + aggT)
        hT = jnp.maximum(_dot3(w1th_ref[...], w1tl_ref[...], uh, ul), 0.0)
        pgT = jnp.dot(w2tb_ref[...], hT.astype(BF), preferred_element_type=f32)
        p0, p1 = pgT[0:1, :], pgT[1:2, :]
        pm = jnp.maximum(p0, p1)
        e0, e1 = jnp.exp(p0 - pm), jnp.exp(p1 - pm)
        keep = e1 / (e0 + e1)

        li = _dot3(xh_ref[b], xl_ref[b], wih_ref[...], wil_ref[...]) + bi_ref[0:1, :]
        li_ref[b] = li
        liT = _dot3(with_ref[...], witl_ref[...], xth_ref[b], xtl_ref[b]) + bit_ref[...]
        sT = liT * keep

        idcT = idct_ref[b]
        q = jnp.maximum(sT[0:1, :], sT[1:2, :])
        bestv = jnp.full((1, 1), -INF, f32)
        bestk = jnp.zeros((1, 1), jnp.int32)
        for k in range(K):
            memb = (idcT[:, k:k + 1] == nlane128)
            anyk = jnp.max(memb.astype(f32), axis=0, keepdims=True) > 0.0
            ck = jnp.max(jnp.where(anyk, q, -INF), axis=1, keepdims=True)
            upd = ck > bestv
            bestv = jnp.where(upd, ck, bestv)
            bestk = jnp.where(upd, jnp.full((1, 1), k, jnp.int32), bestk)

        ind = (kio == bestk).astype(jnp.int32)
        selcol = jnp.sum(idcT * ind, axis=1, keepdims=True)
        cnt = jnp.sum((selcol == nlane128).astype(f32), axis=0, keepdims=True)

        pooled = jnp.dot(cnt, xb, preferred_element_type=f32) * (1.0 / M)
        ph, plo = _split(pooled)
        lb_ref[b] = _dot3(ph, plo, whh_ref[...], whl_ref[...]) + bh_ref[0:1, :]


def kernel(x, clusters_idcs, W_gcn1, W_gcn2, W_inst, b_inst, W_head, b_head):
    f32 = jnp.float32
    xp = jnp.zeros((B, NP, D), f32).at[:, :N, :].set(x.astype(f32))
    xt = jnp.swapaxes(xp, 1, 2)
    xh = xp.astype(BF)
    xl = (xp - xh.astype(f32)).astype(BF)
    xth = xt.astype(BF)
    xtl = (xt - xth.astype(f32)).astype(BF)
    idct = jnp.full((B, 128, K), PADIDX, jnp.int32).at[:, :M, :].set(
        jnp.swapaxes(clusters_idcs.astype(jnp.int32), 1, 2))

    def split2(a):
        hi = a.astype(BF)
        return hi, (a - hi.astype(f32)).astype(BF)

    w1t = W_gcn1.astype(f32).T
    w1th, w1tl = split2(w1t)
    w2tb = jnp.zeros((8, D), f32).at[:CC, :].set(W_gcn2.astype(f32).T).astype(BF)
    wit = jnp.zeros((8, D), f32).at[:CC, :].set(W_inst.astype(f32).T)
    with_, witl = split2(wit)
    bit = jnp.zeros((8, 1), f32).at[:CC, 0].set(b_inst.astype(f32))
    wip = jnp.zeros((D, D), f32).at[:, :CC].set(W_inst.astype(f32))
    wih, wil = split2(wip)
    bip = jnp.zeros((8, D), f32).at[0, :CC].set(b_inst.astype(f32))
    whp = jnp.zeros((D, D), f32).at[:, :CC].set(W_head.astype(f32))
    whh, whl = split2(whp)
    bhp = jnp.zeros((8, D), f32).at[0, :CC].set(b_head.astype(f32))

    li, lb = pl.pallas_call(
        _bag_kernel,
        grid=(B // NB,),
        in_specs=[
            pl.BlockSpec((NB, NP, D), lambda i: (i, 0, 0)),
            pl.BlockSpec((NB, D, NP), lambda i: (i, 0, 0)),
            pl.BlockSpec((NB, NP, D), lambda i: (i, 0, 0)),
            pl.BlockSpec((NB, NP, D), lambda i: (i, 0, 0)),
            pl.BlockSpec((NB, D, NP), lambda i: (i, 0, 0)),
            pl.BlockSpec((NB, D, NP), lambda i: (i, 0, 0)),
            pl.BlockSpec((NB, 128, K), lambda i: (i, 0, 0)),
            pl.BlockSpec((D, D), lambda i: (0, 0)),
            pl.BlockSpec((D, D), lambda i: (0, 0)),
            pl.BlockSpec((8, D), lambda i: (0, 0)),
            pl.BlockSpec((8, D), lambda i: (0, 0)),
            pl.BlockSpec((8, D), lambda i: (0, 0)),
            pl.BlockSpec((8, 1), lambda i: (0, 0)),
            pl.BlockSpec((D, D), lambda i: (0, 0)),
            pl.BlockSpec((D, D), lambda i: (0, 0)),
            pl.BlockSpec((8, D), lambda i: (0, 0)),
            pl.BlockSpec((D, D), lambda i: (0, 0)),
            pl.BlockSpec((D, D), lambda i: (0, 0)),
            pl.BlockSpec((8, D), lambda i: (0, 0)),
        ],
        out_specs=[
            pl.BlockSpec((NB, NP, D), lambda i: (i, 0, 0)),
            pl.BlockSpec((NB, 1, D), lambda i: (i, 0, 0)),
        ],
        out_shape=[
            jax.ShapeDtypeStruct((B, NP, D), f32),
            jax.ShapeDtypeStruct((B, 1, D), f32),
        ],
    )(xp, xt, xh, xl, xth, xtl, idct, w1th, w1tl, w2tb, with_, witl,
      bit, wih, wil, bip, whh, whl, bhp)

    return lb[:, 0, :CC], li[:, :N, :CC]
